# baseline (device time: 18111 ns/iter reference)
import jax
import jax.numpy as jnp
from jax import lax
from jax.experimental import pallas as pl
from jax.experimental.pallas import tpu as pltpu

Z = 4

_ORDER = {k: sorted((p for p in range(Z) if p != k), key=lambda p: abs(p - k))
          for k in range(Z)}


def kernel(x):
    m, n = x.shape
    blk = n // Z

    def body(x_ref, out_ref, send_sems, recv_sems, credit, local_sem):
        my_x = lax.axis_index("x")
        my_y = lax.axis_index("y")
        my_z = lax.axis_index("z")

        barrier_sem = pltpu.get_barrier_semaphore()
        pl.semaphore_signal(barrier_sem, inc=1)
        pl.semaphore_wait(barrier_sem, 1)

        diag = pltpu.make_async_copy(
            x_ref.at[:, pl.ds(my_z * blk, blk)],
            out_ref.at[pl.ds(my_z * m, m), :],
            local_sem,
        )
        diag.start()

        for k in range(Z):
            @pl.when(my_z == k)
            def _(k=k):
                order = _ORDER[k]

                for p in order:
                    slot = (k - p) % Z - 1
                    pl.semaphore_signal(
                        credit.at[slot], inc=1,
                        device_id=(my_x, my_y, p),
                        device_id_type=pl.DeviceIdType.MESH,
                    )

                sends = []
                for q in order:
                    myslot = (q - k) % Z - 1
                    pl.semaphore_wait(credit.at[myslot], 1)
                    rdma = pltpu.make_async_remote_copy(
                        src_ref=x_ref.at[:, pl.ds(q * blk, blk)],
                        dst_ref=out_ref.at[pl.ds(k * m, m), :],
                        send_sem=send_sems.at[myslot],
                        recv_sem=recv_sems.at[myslot],
                        device_id=(my_x, my_y, q),
                        device_id_type=pl.DeviceIdType.MESH,
                    )
                    rdma.start()
                    sends.append(rdma)

                for p in order:
                    s = (k - p) % Z - 1
                    rows = pl.ds(p * m, m)
                    pltpu.make_async_remote_copy(
                        src_ref=out_ref.at[rows, :],
                        dst_ref=out_ref.at[rows, :],
                        send_sem=send_sems.at[s],
                        recv_sem=recv_sems.at[s],
                        device_id=(my_x, my_y, p),
                        device_id_type=pl.DeviceIdType.MESH,
                    ).wait_recv()

                for rdma in sends:
                    rdma.wait_send()

        diag.wait()

    out_shape = jax.ShapeDtypeStruct((Z * m, blk), x.dtype)
    return pl.pallas_call(
        body,
        out_shape=out_shape,
        in_specs=[pl.BlockSpec(memory_space=pltpu.HBM)],
        out_specs=pl.BlockSpec(memory_space=pltpu.HBM),
        scratch_shapes=[
            pltpu.SemaphoreType.DMA((Z - 1,)),
            pltpu.SemaphoreType.DMA((Z - 1,)),
            pltpu.SemaphoreType.REGULAR((Z - 1,)),
            pltpu.SemaphoreType.DMA,
        ],
        compiler_params=pltpu.CompilerParams(collective_id=0),
    )(x)


# device time: 16152 ns/iter; 1.1213x vs baseline; 1.1213x over previous
import jax
import jax.numpy as jnp
from jax import lax
from jax.experimental import pallas as pl
from jax.experimental.pallas import tpu as pltpu

Z = 4


def kernel(x):
    m, n = x.shape
    blk = n // Z

    def body(x_ref, out_ref, send_sems, recv_sems, local_sem):
        my_x = lax.axis_index("x")
        my_y = lax.axis_index("y")
        my_z = lax.axis_index("z")

        barrier_sem = pltpu.get_barrier_semaphore()
        for r in range(1, Z):
            pl.semaphore_signal(
                barrier_sem, inc=1,
                device_id=(my_x, my_y, (my_z + r) % Z),
                device_id_type=pl.DeviceIdType.MESH,
            )
        pl.semaphore_wait(barrier_sem, Z - 1)

        diag = pltpu.make_async_copy(
            x_ref.at[:, pl.ds(my_z * blk, blk)],
            out_ref.at[pl.ds(my_z * m, m), :],
            local_sem,
        )
        diag.start()

        rdmas = []
        for r in range(1, Z):
            tgt = (my_z + r) % Z
            rdma = pltpu.make_async_remote_copy(
                src_ref=x_ref.at[:, pl.ds(tgt * blk, blk)],
                dst_ref=out_ref.at[pl.ds(my_z * m, m), :],
                send_sem=send_sems.at[r - 1],
                recv_sem=recv_sems.at[r - 1],
                device_id=(my_x, my_y, tgt),
                device_id_type=pl.DeviceIdType.MESH,
            )
            rdma.start()
            rdmas.append(rdma)

        diag.wait()
        for rdma in rdmas:
            rdma.wait()

    out_shape = jax.ShapeDtypeStruct((Z * m, blk), x.dtype)
    return pl.pallas_call(
        body,
        out_shape=out_shape,
        in_specs=[pl.BlockSpec(memory_space=pltpu.HBM)],
        out_specs=pl.BlockSpec(memory_space=pltpu.HBM),
        scratch_shapes=[
            pltpu.SemaphoreType.DMA((Z - 1,)),
            pltpu.SemaphoreType.DMA((Z - 1,)),
            pltpu.SemaphoreType.DMA,
        ],
        compiler_params=pltpu.CompilerParams(collective_id=0),
    )(x)
